# baseline (speedup 1.0000x reference)
import jax
import jax.numpy as jnp
from jax.experimental import pallas as pl
from jax.experimental.pallas import tpu as pltpu

E = 8
R = 8
D = 768
DFF = 128
GATE_THRESHOLD = 0.5
GATE_TEMPERATURE = 1.0

_T = 512


def _moe_block_kernel(x_ref, wa_ref, wg_ref, wu_ref, wd_ref, sb_ref,
                      out_ref, gs_ref):
    x = x_ref[...].astype(jnp.bfloat16)

    gh = jax.lax.dot_general(x, wa_ref[...], (((1,), (1,)), ((), ())),
                             preferred_element_type=jnp.float32)
    gh2 = gh * gh
    row = jax.lax.broadcasted_iota(jnp.int32, (E * R, E), 0)
    col = jax.lax.broadcasted_iota(jnp.int32, (E * R, E), 1)
    group = (row // R == col).astype(jnp.float32)
    gh2_hi = gh2.astype(jnp.bfloat16).astype(jnp.float32)
    gh2_lo = gh2 - gh2_hi
    dot = lambda a, b: jax.lax.dot_general(
        a, b, (((1,), (0,)), ((), ())), preferred_element_type=jnp.float32)
    ss = dot(gh2_hi, group) + dot(gh2_lo, group)
    scores = jnp.sqrt(ss * (1.0 / R) + 1e-6)

    threshold = GATE_THRESHOLD / GATE_TEMPERATURE
    s = scores * sb_ref[0:1, :] - sb_ref[1:2, :]
    m = s >= threshold
    sm = jnp.where(m, s, 0.0)
    gs_ref[...] = jnp.where(m, s, -jnp.inf)

    hg = jax.lax.dot_general(x, wg_ref[...], (((1,), (1,)), ((), ())),
                             preferred_element_type=jnp.float32)
    hu = jax.lax.dot_general(x, wu_ref[...], (((1,), (1,)), ((), ())),
                             preferred_element_type=jnp.float32)
    h = (hg * jax.lax.logistic(hg)) * hu

    erow = jax.lax.broadcasted_iota(jnp.int32, (E, E * DFF), 0)
    ecol = jax.lax.broadcasted_iota(jnp.int32, (E, E * DFF), 1)
    expand = (ecol // DFF == erow).astype(jnp.float32)
    sm_hi = sm.astype(jnp.bfloat16).astype(jnp.float32)
    sm_lo = sm - sm_hi
    dot2 = lambda a, b: jax.lax.dot_general(
        a, b, (((1,), (0,)), ((), ())), preferred_element_type=jnp.float32)
    sm_big = dot2(sm_hi, expand) + dot2(sm_lo, expand)
    hs = h * sm_big

    out_ref[...] = jax.lax.dot_general(hs, wd_ref[...],
                                       (((1,), (0,)), ((), ())),
                                       preferred_element_type=jnp.float32)


def kernel(hidden_states, W_A, W_gate, W_up, W_down, gate_scale, gate_bias):
    orig_shape = hidden_states.shape
    x = hidden_states.reshape(-1, orig_shape[-1])
    N = x.shape[0]

    wa = W_A.reshape(E * R, D).astype(jnp.bfloat16)
    wg = W_gate.reshape(E * DFF, D).astype(jnp.bfloat16)
    wu = W_up.reshape(E * DFF, D).astype(jnp.bfloat16)
    wd = jnp.transpose(W_down, (0, 2, 1)).reshape(E * DFF, D).astype(jnp.bfloat16)
    sb = jnp.stack([gate_scale, gate_bias], axis=0)

    grid = (N // _T,)
    out, gs = pl.pallas_call(
        _moe_block_kernel,
        grid=grid,
        in_specs=[
            pl.BlockSpec((_T, D), lambda i: (i, 0)),
            pl.BlockSpec((E * R, D), lambda i: (0, 0)),
            pl.BlockSpec((E * DFF, D), lambda i: (0, 0)),
            pl.BlockSpec((E * DFF, D), lambda i: (0, 0)),
            pl.BlockSpec((E * DFF, D), lambda i: (0, 0)),
            pl.BlockSpec((2, E), lambda i: (0, 0)),
        ],
        out_specs=[
            pl.BlockSpec((_T, D), lambda i: (i, 0)),
            pl.BlockSpec((_T, E), lambda i: (i, 0)),
        ],
        out_shape=[
            jax.ShapeDtypeStruct((N, D), jnp.float32),
            jax.ShapeDtypeStruct((N, E), jnp.float32),
        ],
    )(x, wa, wg, wu, wd, sb)

    return out.reshape(orig_shape), gs.reshape(orig_shape[:-1] + (E,))


# speedup vs baseline: 1.1199x; 1.1199x over previous
import jax
import jax.numpy as jnp
from jax.experimental import pallas as pl
from jax.experimental.pallas import tpu as pltpu

E = 8
R = 8
D = 768
DFF = 128
GATE_THRESHOLD = 0.5
GATE_TEMPERATURE = 1.0

_T = 1024


def _moe_block_kernel(x_ref, wa_ref, wg_ref, wu_ref, wd_ref, sb_ref,
                      out_ref, gs_ref):
    x = x_ref[...].astype(jnp.bfloat16)

    gh = jax.lax.dot_general(x, wa_ref[...], (((1,), (1,)), ((), ())),
                             preferred_element_type=jnp.float32)
    gh2 = gh * gh
    row = jax.lax.broadcasted_iota(jnp.int32, (E * R, E), 0)
    col = jax.lax.broadcasted_iota(jnp.int32, (E * R, E), 1)
    group = (row // R == col).astype(jnp.float32)
    gh2_hi = gh2.astype(jnp.bfloat16).astype(jnp.float32)
    gh2_lo = gh2 - gh2_hi
    dot = lambda a, b: jax.lax.dot_general(
        a, b, (((1,), (0,)), ((), ())), preferred_element_type=jnp.float32)
    ss = dot(gh2_hi, group) + dot(gh2_lo, group)
    scores = jnp.sqrt(ss * (1.0 / R) + 1e-6)

    threshold = GATE_THRESHOLD / GATE_TEMPERATURE
    s = scores * sb_ref[0:1, :] - sb_ref[1:2, :]
    m = s >= threshold
    sm = jnp.where(m, s, 0.0)
    gs_ref[...] = jnp.where(m, s, -jnp.inf)

    hg = jax.lax.dot_general(x, wg_ref[...], (((1,), (1,)), ((), ())),
                             preferred_element_type=jnp.float32)
    hu = jax.lax.dot_general(x, wu_ref[...], (((1,), (1,)), ((), ())),
                             preferred_element_type=jnp.float32)
    h = (hg * jax.lax.logistic(hg)) * hu

    erow = jax.lax.broadcasted_iota(jnp.int32, (2 * E, E * DFF), 0)
    ecol = jax.lax.broadcasted_iota(jnp.int32, (2 * E, E * DFF), 1)
    expand = (ecol // DFF == erow % E).astype(jnp.float32)
    sm_hi = sm.astype(jnp.bfloat16).astype(jnp.float32)
    sm_lo = sm - sm_hi
    sm2 = jnp.concatenate([sm_hi, sm_lo], axis=1)
    sm_big = jax.lax.dot_general(sm2, expand, (((1,), (0,)), ((), ())),
                                 preferred_element_type=jnp.float32)
    hs = (h * sm_big).astype(jnp.bfloat16)

    out_ref[...] = jax.lax.dot_general(hs, wd_ref[...],
                                       (((1,), (0,)), ((), ())),
                                       preferred_element_type=jnp.float32)


def kernel(hidden_states, W_A, W_gate, W_up, W_down, gate_scale, gate_bias):
    orig_shape = hidden_states.shape
    x = hidden_states.reshape(-1, orig_shape[-1])
    N = x.shape[0]

    wa = W_A.reshape(E * R, D).astype(jnp.bfloat16)
    wg = W_gate.reshape(E * DFF, D).astype(jnp.bfloat16)
    wu = W_up.reshape(E * DFF, D).astype(jnp.bfloat16)
    wd = jnp.transpose(W_down, (0, 2, 1)).reshape(E * DFF, D).astype(jnp.bfloat16)
    sb = jnp.stack([gate_scale, gate_bias], axis=0)

    grid = (N // _T,)
    out, gs = pl.pallas_call(
        _moe_block_kernel,
        grid=grid,
        in_specs=[
            pl.BlockSpec((_T, D), lambda i: (i, 0)),
            pl.BlockSpec((E * R, D), lambda i: (0, 0)),
            pl.BlockSpec((E * DFF, D), lambda i: (0, 0)),
            pl.BlockSpec((E * DFF, D), lambda i: (0, 0)),
            pl.BlockSpec((E * DFF, D), lambda i: (0, 0)),
            pl.BlockSpec((2, E), lambda i: (0, 0)),
        ],
        out_specs=[
            pl.BlockSpec((_T, D), lambda i: (i, 0)),
            pl.BlockSpec((_T, E), lambda i: (i, 0)),
        ],
        out_shape=[
            jax.ShapeDtypeStruct((N, D), jnp.float32),
            jax.ShapeDtypeStruct((N, E), jnp.float32),
        ],
    )(x, wa, wg, wu, wd, sb)

    return out.reshape(orig_shape), gs.reshape(orig_shape[:-1] + (E,))


# speedup vs baseline: 1.2672x; 1.1315x over previous
import jax
import jax.numpy as jnp
from jax.experimental import pallas as pl
from jax.experimental.pallas import tpu as pltpu

E = 8
R = 8
D = 768
DFF = 128
GATE_THRESHOLD = 0.5
GATE_TEMPERATURE = 1.0

_T = 2048
_NSUB = 2


def _prep_weights(wg_ref, wu_ref, wdr_ref, wgu_bf, wd_bf):
    @pl.when(pl.program_id(0) == 0)
    def _():
        wgu_bf[0:E * DFF, :] = wg_ref[...].astype(jnp.bfloat16)
        wgu_bf[E * DFF:, :] = wu_ref[...].astype(jnp.bfloat16)
        for e in range(E):
            chunk = wdr_ref[pl.ds(e * D, D), :]
            wd_bf[pl.ds(e * DFF, DFF), :] = jnp.transpose(
                chunk, (1, 0)).astype(jnp.bfloat16)


def _moe_sub_block(x_ref, wa_ref, scale_ref, bias_ref,
                   wgu_bf, wd_bf, out_ref, gs_ref, sub):
    TS = _T // _NSUB
    x = x_ref[pl.ds(sub * TS, TS), :].astype(jnp.bfloat16)

    gh = jax.lax.dot_general(x, wa_ref[...].astype(jnp.bfloat16),
                             (((1,), (1,)), ((), ())),
                             preferred_element_type=jnp.float32)
    gh2 = gh * gh
    row = jax.lax.broadcasted_iota(jnp.int32, (2 * E * R, E), 0)
    col = jax.lax.broadcasted_iota(jnp.int32, (2 * E * R, E), 1)
    group = ((row % (E * R)) // R == col).astype(jnp.float32)
    gh2_hi = gh2.astype(jnp.bfloat16).astype(jnp.float32)
    gh2_lo = gh2 - gh2_hi
    gh2_2 = jnp.concatenate([gh2_hi, gh2_lo], axis=1)
    ss = jax.lax.dot_general(gh2_2, group, (((1,), (0,)), ((), ())),
                             preferred_element_type=jnp.float32)
    scores = jnp.sqrt(ss * (1.0 / R) + 1e-6)

    threshold = GATE_THRESHOLD / GATE_TEMPERATURE
    s = scores * scale_ref[...] - bias_ref[...]
    m = s >= threshold
    sm = jnp.where(m, s, 0.0).astype(jnp.bfloat16)
    gs_ref[pl.ds(sub * TS, TS), :] = jnp.where(m, s, -jnp.inf)

    hgu = jax.lax.dot_general(x, wgu_bf[...], (((1,), (1,)), ((), ())),
                              preferred_element_type=jnp.float32
                              ).astype(jnp.bfloat16)
    hg = hgu[:, :E * DFF]
    hu = hgu[:, E * DFF:]
    h = (hg * jax.lax.logistic(hg)) * hu

    erow = jax.lax.broadcasted_iota(jnp.int32, (E, E * DFF), 0)
    ecol = jax.lax.broadcasted_iota(jnp.int32, (E, E * DFF), 1)
    expand = (ecol // DFF == erow).astype(jnp.bfloat16)
    sm_big = jax.lax.dot_general(sm, expand, (((1,), (0,)), ((), ())),
                                 preferred_element_type=jnp.float32
                                 ).astype(jnp.bfloat16)
    hs = h * sm_big

    out_ref[pl.ds(sub * TS, TS), :] = jax.lax.dot_general(
        hs, wd_bf[...], (((1,), (0,)), ((), ())),
        preferred_element_type=jnp.float32)


def _moe_block_kernel(x_ref, wa_ref, wg_ref, wu_ref, wdr_ref,
                      scale_ref, bias_ref, out_ref, gs_ref,
                      wgu_bf, wd_bf):
    _prep_weights(wg_ref, wu_ref, wdr_ref, wgu_bf, wd_bf)
    for sub in range(_NSUB):
        _moe_sub_block(x_ref, wa_ref, scale_ref, bias_ref,
                       wgu_bf, wd_bf, out_ref, gs_ref, sub)


def kernel(hidden_states, W_A, W_gate, W_up, W_down, gate_scale, gate_bias):
    orig_shape = hidden_states.shape
    x = hidden_states.reshape(-1, orig_shape[-1])
    N = x.shape[0]

    wa = W_A.reshape(E * R, D)
    wg = W_gate.reshape(E * DFF, D)
    wu = W_up.reshape(E * DFF, D)
    wdr = W_down.reshape(E * D, DFF)
    scale = gate_scale.reshape(1, E)
    bias = gate_bias.reshape(1, E)

    grid = (N // _T,)
    out, gs = pl.pallas_call(
        _moe_block_kernel,
        grid=grid,
        in_specs=[
            pl.BlockSpec((_T, D), lambda i: (i, 0)),
            pl.BlockSpec((E * R, D), lambda i: (0, 0)),
            pl.BlockSpec((E * DFF, D), lambda i: (0, 0)),
            pl.BlockSpec((E * DFF, D), lambda i: (0, 0)),
            pl.BlockSpec((E * D, DFF), lambda i: (0, 0)),
            pl.BlockSpec((1, E), lambda i: (0, 0)),
            pl.BlockSpec((1, E), lambda i: (0, 0)),
        ],
        out_specs=[
            pl.BlockSpec((_T, D), lambda i: (i, 0)),
            pl.BlockSpec((_T, E), lambda i: (i, 0)),
        ],
        out_shape=[
            jax.ShapeDtypeStruct((N, D), jnp.float32),
            jax.ShapeDtypeStruct((N, E), jnp.float32),
        ],
        scratch_shapes=[
            pltpu.VMEM((2 * E * DFF, D), jnp.bfloat16),
            pltpu.VMEM((E * DFF, D), jnp.bfloat16),
        ],
    )(x, wa, wg, wu, wdr, scale, bias)

    return out.reshape(orig_shape), gs.reshape(orig_shape[:-1] + (E,))
